# baseline (speedup 1.0000x reference)
import jax
import jax.numpy as jnp
from jax import lax
from jax.experimental import pallas as pl
from jax.experimental.pallas import tpu as pltpu
from jax.experimental.pallas import tpu_sc as plsc

_N = 100000
_D = 128
_P = 50000

_NT = 16
_N_PAD = 102400
_RPT = _N_PAD // _NT
_CHUNK = 128
_NCH = 25
_P_PAD = _NT * _NCH * _CHUNK


def _mask_sc_kernel(idx_hbm, zeros_hbm, mask_hbm, idx_v, ones_v, sem):
    tid = lax.axis_index("s")

    for i in range(_CHUNK // 16):
        ones_v[pl.ds(i * 16, 16)] = jnp.ones((16,), jnp.float32)

    zc = pltpu.async_copy(zeros_hbm, mask_hbm.at[pl.ds(tid * _RPT, _RPT)], sem)
    ic = pltpu.async_copy(idx_hbm.at[tid], idx_v, sem)
    zc.wait()
    ic.wait()
    plsc.subcore_barrier()

    copies = [
        pltpu.async_copy(ones_v, mask_hbm.at[idx_v.at[j]], sem)
        for j in range(_NCH)
    ]
    for c in copies:
        c.wait()


def _build_mask(idx3, zeros):
    mesh = plsc.VectorSubcoreMesh(
        core_axis_name="c", subcore_axis_name="s", num_cores=1)
    k = pl.kernel(
        _mask_sc_kernel,
        out_type=jax.ShapeDtypeStruct((_N_PAD,), jnp.float32),
        mesh=mesh,
        scratch_types=[
            pltpu.VMEM((_NCH, _CHUNK), jnp.int32),
            pltpu.VMEM((_CHUNK,), jnp.float32),
            pltpu.SemaphoreType.DMA,
        ],
    )
    return k(idx3, zeros)


def _update_tc_kernel(x_ref, w_ref, b_ref, m_ref, o_ref):
    x = x_ref[...]
    y = lax.dot_general(x, w_ref[...], (((1,), (1,)), ((), ())),
                        preferred_element_type=jnp.float32) + b_ref[...]
    o_ref[...] = jnp.where(m_ref[...] > 0.0, y, x)


_BLK = 4000


def kernel(node_tensor, partition_idx, W, b):
    idx = partition_idx.astype(jnp.int32)
    pad = jnp.broadcast_to(idx[:1], (_P_PAD - _P,))
    idx3 = jnp.concatenate([idx, pad]).reshape(_NT, _NCH, _CHUNK)
    zeros = jnp.zeros((_RPT,), jnp.float32)
    mask = _build_mask(idx3, zeros).reshape(_N_PAD, 1)
    out = pl.pallas_call(
        _update_tc_kernel,
        grid=(_N // _BLK,),
        in_specs=[
            pl.BlockSpec((_BLK, _D), lambda i: (i, 0)),
            pl.BlockSpec((_D, _D), lambda i: (0, 0)),
            pl.BlockSpec((1, _D), lambda i: (0, 0)),
            pl.BlockSpec((_BLK, 1), lambda i: (i, 0)),
        ],
        out_specs=pl.BlockSpec((_BLK, _D), lambda i: (i, 0)),
        out_shape=jax.ShapeDtypeStruct((_N, _D), jnp.float32),
    )(node_tensor, W, b.reshape(1, _D), mask)
    return out
